# baseline (speedup 1.0000x reference)
import functools

import jax
import jax.numpy as jnp
from jax import lax
from jax.experimental import pallas as pl
from jax.experimental.pallas import tpu as pltpu
from jax.experimental.pallas import tpu_sc as plsc

E = 2048
NS = 16
L = 16
EDGES_PER_TILE = E // NS
NV = EDGES_PER_TILE // L


def _ncut_body(ei_hbm, w_hbm, out_hbm, fin_hbm, rows_v, cols_v, w_v,
               bins_v, acc_v, gat_v, res_v, sem_r, sem_c, sem_w):
    s = lax.axis_index("s")
    lane = lax.iota(jnp.int32, L)
    zero = jnp.zeros((L,), jnp.float32)

    base = s * EDGES_PER_TILE
    cp_r = pltpu.async_copy(ei_hbm.at[0, pl.ds(base, EDGES_PER_TILE)], rows_v, sem_r)
    cp_c = pltpu.async_copy(ei_hbm.at[1, pl.ds(base, EDGES_PER_TILE)], cols_v, sem_c)
    cp_w = pltpu.async_copy(w_hbm.at[pl.ds(base, EDGES_PER_TILE)], w_v, sem_w)

    def zrow(i, carry):
        bins_v[i, :] = zero
        return carry

    lax.fori_loop(0, L, zrow, 0)
    cp_r.wait()
    cp_c.wait()
    cp_w.wait()

    def step(j, carry):
        off = j * L
        r = rows_v[pl.ds(off, L)]
        cl = cols_v[pl.ds(off, L)]
        w = w_v[pl.ds(off, L)]
        kr = lax.shift_right_logical(r, 3)
        kc = lax.shift_right_logical(cl, 3)
        same = kr == kc
        plsc.addupdate_scatter(bins_v, [lane, kr], w, mask=jnp.logical_not(same))
        plsc.addupdate_scatter(bins_v, [lane, kc], jnp.where(same, w + w, w))
        plsc.addupdate_scatter(bins_v, [lane, kr + 8], w, mask=same)
        return carry

    lax.fori_loop(0, NV, step, 0)

    def rrow(i, t):
        return t + bins_v[i, :]

    part = lax.fori_loop(0, L, rrow, zero)
    acc_v[...] = part

    @pl.when(s != 0)
    def _publish():
        pltpu.sync_copy(acc_v, out_hbm.at[pl.ds(s * L, L)])

    plsc.subcore_barrier()

    @pl.when(s == 0)
    def _finalize():
        pltpu.sync_copy(out_hbm.at[pl.ds(L, (NS - 1) * L)], gat_v)
        total = part
        for i in range(NS - 1):
            total = total + gat_v[pl.ds(i * L, L)]
        res_v[...] = total
        num = 2.0 * plsc.load_gather(res_v, [jnp.bitwise_and(lane + 8, 15)])
        ratio = jnp.where(lane < 8, num / total, 0.0)
        out = 1.0 - jnp.sum(ratio) * 0.125
        acc_v[...] = jnp.where(lane == 0, out, 0.0)
        pltpu.sync_copy(acc_v.at[pl.ds(0, 1)], fin_hbm)


@functools.partial(
    pl.kernel,
    out_type=(jax.ShapeDtypeStruct((NS * L,), jnp.float32),
              jax.ShapeDtypeStruct((1,), jnp.float32)),
    mesh=plsc.VectorSubcoreMesh(core_axis_name="c", subcore_axis_name="s",
                                num_cores=1, num_subcores=16),
    scratch_types=[
        pltpu.VMEM((EDGES_PER_TILE,), jnp.int32),
        pltpu.VMEM((EDGES_PER_TILE,), jnp.int32),
        pltpu.VMEM((EDGES_PER_TILE,), jnp.float32),
        pltpu.VMEM((L, L), jnp.float32),
        pltpu.VMEM((L,), jnp.float32),
        pltpu.VMEM(((NS - 1) * L,), jnp.float32),
        pltpu.VMEM((L,), jnp.float32),
        pltpu.SemaphoreType.DMA,
        pltpu.SemaphoreType.DMA,
        pltpu.SemaphoreType.DMA,
    ],
    compiler_params=pltpu.CompilerParams(needs_layout_passes=False),
)
def _ncut_sc(ei_hbm, w_hbm, out_hbm, fin_hbm, rows_v, cols_v, w_v,
             bins_v, acc_v, gat_v, res_v, sem_r, sem_c, sem_w):
    _ncut_body(ei_hbm, w_hbm, out_hbm, fin_hbm, rows_v, cols_v, w_v,
               bins_v, acc_v, gat_v, res_v, sem_r, sem_c, sem_w)


def kernel(edge_index, weight):
    _, fin = _ncut_sc(edge_index, weight)
    return fin.reshape(())


# speedup vs baseline: 1.0102x; 1.0102x over previous
import functools

import jax
import jax.numpy as jnp
from jax import lax
from jax.experimental import pallas as pl
from jax.experimental.pallas import tpu as pltpu
from jax.experimental.pallas import tpu_sc as plsc

E = 2048
NS = 16
L = 16
EDGES_PER_TILE = E // NS
NV = EDGES_PER_TILE // L


def _ncut_body(ei_hbm, w_hbm, out_hbm, fin_hbm, rows_v, cols_v, w_v,
               bins_v, acc_v, gat_v, res_v, sem_r, sem_c, sem_w):
    s = lax.axis_index("s")
    lane = lax.iota(jnp.int32, L)
    zero = jnp.zeros((L,), jnp.float32)

    base = s * EDGES_PER_TILE
    cp_r = pltpu.async_copy(ei_hbm.at[0, pl.ds(base, EDGES_PER_TILE)], rows_v, sem_r)
    cp_c = pltpu.async_copy(ei_hbm.at[1, pl.ds(base, EDGES_PER_TILE)], cols_v, sem_c)
    cp_w = pltpu.async_copy(w_hbm.at[pl.ds(base, EDGES_PER_TILE)], w_v, sem_w)

    def zrow(i, carry):
        bins_v[i, :] = zero
        return carry

    lax.fori_loop(0, L, zrow, 0)
    cp_r.wait()
    cp_c.wait()
    cp_w.wait()

    def step(j, carry):
        off = j * L
        r = rows_v[pl.ds(off, L)]
        cl = cols_v[pl.ds(off, L)]
        w = w_v[pl.ds(off, L)]
        kr = lax.shift_right_logical(r, 3)
        kc = lax.shift_right_logical(cl, 3)
        same = kr == kc
        plsc.addupdate_scatter(bins_v, [lane, kr], w, mask=jnp.logical_not(same))
        plsc.addupdate_scatter(bins_v, [lane, kc], jnp.where(same, w + w, w))
        plsc.addupdate_scatter(bins_v, [lane, kr + 8], w, mask=same)
        return carry

    lax.fori_loop(0, NV, step, 0)

    def rrow(i, t):
        return t + bins_v[i, :]

    part = lax.fori_loop(0, L, rrow, zero)
    acc_v[...] = part

    @pl.when(s != 0)
    def _publish():
        pltpu.sync_copy(acc_v, out_hbm.at[pl.ds(s * L, L)])

    plsc.subcore_barrier()

    @pl.when(s == 0)
    def _finalize():
        pltpu.sync_copy(out_hbm.at[pl.ds(L, (NS - 1) * L)], gat_v)
        total = part
        for i in range(NS - 1):
            total = total + gat_v[pl.ds(i * L, L)]
        res_v[...] = total
        num = 2.0 * plsc.load_gather(res_v, [jnp.bitwise_and(lane + 8, 15)])
        ratio = jnp.where(lane < 8, num / total, 0.0)
        out = 1.0 - jnp.sum(ratio) * 0.125
        acc_v[...] = jnp.where(lane == 0, out, 0.0)
        pltpu.sync_copy(acc_v.at[pl.ds(0, 1)], fin_hbm)


@functools.partial(
    pl.kernel,
    out_type=(jax.ShapeDtypeStruct((NS * L,), jnp.float32),
              jax.ShapeDtypeStruct((1,), jnp.float32)),
    mesh=plsc.VectorSubcoreMesh(core_axis_name="c", subcore_axis_name="s",
                                num_cores=1, num_subcores=16),
    scratch_types=[
        pltpu.VMEM((EDGES_PER_TILE,), jnp.int32),
        pltpu.VMEM((EDGES_PER_TILE,), jnp.int32),
        pltpu.VMEM((EDGES_PER_TILE,), jnp.float32),
        pltpu.VMEM((L, L), jnp.float32),
        pltpu.VMEM((L,), jnp.float32),
        pltpu.VMEM(((NS - 1) * L,), jnp.float32),
        pltpu.VMEM((L,), jnp.float32),
        pltpu.SemaphoreType.DMA,
        pltpu.SemaphoreType.DMA,
        pltpu.SemaphoreType.DMA,
    ],
    compiler_params=pltpu.CompilerParams(needs_layout_passes=False, use_tc_tiling_on_sc=False),
)
def _ncut_sc(ei_hbm, w_hbm, out_hbm, fin_hbm, rows_v, cols_v, w_v,
             bins_v, acc_v, gat_v, res_v, sem_r, sem_c, sem_w):
    _ncut_body(ei_hbm, w_hbm, out_hbm, fin_hbm, rows_v, cols_v, w_v,
               bins_v, acc_v, gat_v, res_v, sem_r, sem_c, sem_w)


def kernel(edge_index, weight):
    _, fin = _ncut_sc(edge_index, weight)
    return fin.reshape(())
